# baseline (speedup 1.0000x reference)
import functools

import jax
import jax.numpy as jnp
from jax import lax
from jax.experimental import pallas as pl
from jax.experimental.pallas import tpu as pltpu
from jax.experimental.pallas import tpu_sc as plsc

VOCAB = 1000000
EMB_DIM = 32
BATCH = 4096
SEQ_LEN = 50
TOTAL = BATCH * SEQ_LEN

NC, NS = 2, 16
NW = NC * NS
B_PER_W = TOTAL // NW
CHUNK = 1600
NCHUNK = B_PER_W // CHUNK


def _make_kernel():
    mesh = plsc.VectorSubcoreMesh(core_axis_name="c", subcore_axis_name="s")

    @functools.partial(
        pl.kernel,
        mesh=mesh,
        out_type=jax.ShapeDtypeStruct((TOTAL, EMB_DIM), jnp.float32),
        scratch_types=[
            pltpu.VMEM((CHUNK,), jnp.int32),
            pltpu.VMEM((CHUNK, EMB_DIM), jnp.float32),
            pltpu.SemaphoreType.DMA,
        ],
        compiler_params=pltpu.CompilerParams(use_tc_tiling_on_sc=False),
    )
    def gather_kernel(idx_hbm, table_hbm, out_hbm, idx_v, rows_v, sem):
        wid = lax.axis_index("s") * NC + lax.axis_index("c")
        base = wid * B_PER_W
        for c in range(NCHUNK):
            off = base + c * CHUNK
            pltpu.sync_copy(idx_hbm.at[pl.ds(off, CHUNK)], idx_v)
            pltpu.async_copy(table_hbm.at[idx_v], rows_v, sem).wait()
            pltpu.sync_copy(rows_v, out_hbm.at[pl.ds(off, CHUNK)])

    return gather_kernel


_gather = _make_kernel()


@jax.jit
def kernel(indices, table):
    flat_idx = indices.reshape(TOTAL).astype(jnp.int32)
    out = _gather(flat_idx, table)
    return out.reshape(BATCH, SEQ_LEN, EMB_DIM)


# speedup vs baseline: 1.0023x; 1.0023x over previous
import functools

import jax
import jax.numpy as jnp
from jax import lax
from jax.experimental import pallas as pl
from jax.experimental.pallas import tpu as pltpu
from jax.experimental.pallas import tpu_sc as plsc

VOCAB = 1000000
EMB_DIM = 32
BATCH = 4096
SEQ_LEN = 50
TOTAL = BATCH * SEQ_LEN

NC, NS = 2, 16
NW = NC * NS
B_PER_W = TOTAL // NW
CHUNK = 800
NCHUNK = B_PER_W // CHUNK
NBUF = 2


def _make_kernel():
    mesh = plsc.VectorSubcoreMesh(core_axis_name="c", subcore_axis_name="s")

    @functools.partial(
        pl.kernel,
        mesh=mesh,
        out_type=jax.ShapeDtypeStruct((TOTAL, EMB_DIM), jnp.float32),
        scratch_types=[
            pltpu.VMEM((NBUF, CHUNK), jnp.int32),
            pltpu.VMEM((NBUF, CHUNK, EMB_DIM), jnp.float32),
            [pltpu.SemaphoreType.DMA] * NBUF,
            [pltpu.SemaphoreType.DMA] * NBUF,
            [pltpu.SemaphoreType.DMA] * NBUF,
        ],
        compiler_params=pltpu.CompilerParams(use_tc_tiling_on_sc=False),
    )
    def gather_kernel(idx_hbm, table_hbm, out_hbm, idx_v, rows_v,
                      isem, gsem, osem):
        wid = lax.axis_index("s") * NC + lax.axis_index("c")
        base = wid * B_PER_W

        i_h = [None] * NBUF
        g_h = [None] * NBUF
        o_h = [None] * NBUF
        for b in range(NBUF):
            i_h[b] = pltpu.async_copy(
                idx_hbm.at[pl.ds(base + b * CHUNK, CHUNK)], idx_v.at[b],
                isem[b])
        for c in range(NCHUNK + 1):
            if c < NCHUNK:
                b = c % NBUF
                i_h[b].wait()
                if o_h[b] is not None:
                    o_h[b].wait()
                g_h[b] = pltpu.async_copy(
                    table_hbm.at[idx_v.at[b]], rows_v.at[b], gsem[b])
            if c >= 1:
                pc = c - 1
                pb = pc % NBUF
                g_h[pb].wait()
                o_h[pb] = pltpu.async_copy(
                    rows_v.at[pb],
                    out_hbm.at[pl.ds(base + pc * CHUNK, CHUNK)], osem[pb])
                if pc + NBUF < NCHUNK:
                    i_h[pb] = pltpu.async_copy(
                        idx_hbm.at[pl.ds(base + (pc + NBUF) * CHUNK, CHUNK)],
                        idx_v.at[pb], isem[pb])
        for b in range(NBUF):
            if o_h[b] is not None:
                o_h[b].wait()

    return gather_kernel


_gather = _make_kernel()


@jax.jit
def kernel(indices, table):
    flat_idx = indices.reshape(TOTAL).astype(jnp.int32)
    out = _gather(flat_idx, table)
    return out.reshape(BATCH, SEQ_LEN, EMB_DIM)


# speedup vs baseline: 1.1561x; 1.1534x over previous
import functools

import jax
import jax.numpy as jnp
from jax import lax
from jax.experimental import pallas as pl
from jax.experimental.pallas import tpu as pltpu
from jax.experimental.pallas import tpu_sc as plsc

VOCAB = 1000000
EMB_DIM = 32
BATCH = 4096
SEQ_LEN = 50

NC, NS = 2, 16
NW = NC * NS
BTILE = BATCH // NW
NGRP = EMB_DIM // 8


def _make_kernel():
    mesh = plsc.VectorSubcoreMesh(core_axis_name="c", subcore_axis_name="s")

    @functools.partial(
        pl.kernel,
        mesh=mesh,
        out_type=jax.ShapeDtypeStruct((SEQ_LEN, NGRP, NW, 8, BTILE),
                                      jnp.float32),
        scratch_types=[
            pltpu.VMEM((SEQ_LEN, BTILE), jnp.int32),
            pltpu.VMEM((2, BTILE, EMB_DIM), jnp.float32),
            pltpu.VMEM((2, NGRP, 8, BTILE), jnp.float32),
            [pltpu.SemaphoreType.DMA] * 2,
            [pltpu.SemaphoreType.DMA] * 2,
        ],
        compiler_params=pltpu.CompilerParams(use_tc_tiling_on_sc=False,
                                             needs_layout_passes=False),
    )
    def gather_kernel(idx_hbm, table_hbm, out_hbm, idx_v, rows_v, out_v,
                      gsem, osem):
        w = lax.axis_index("s") * NC + lax.axis_index("c")
        b0 = w * BTILE
        pltpu.sync_copy(idx_hbm.at[:, pl.ds(b0, BTILE)], idx_v)

        lane = lax.iota(jnp.int32, 16)
        row_ids = [lane + (16 * h) for h in range(BTILE // 16)]

        def gather_s(s, buf):
            return pltpu.make_async_copy(
                table_hbm.at[idx_v.at[s]], rows_v.at[buf], gsem[buf])

        def out_s(s, buf):
            return pltpu.make_async_copy(
                out_v.at[buf], out_hbm.at[s, :, w], osem[buf])

        def transpose(buf):
            for g in range(NGRP):
                for ci in range(8):
                    col = jnp.full((16,), g * 8 + ci, jnp.int32)
                    for h in range(BTILE // 16):
                        vals = plsc.load_gather(rows_v.at[buf],
                                                [row_ids[h], col])
                        out_v[buf, g, ci, pl.ds(16 * h, 16)] = vals

        gather_s(0, 0).start()

        def body(t, carry):
            s0 = 2 * t
            gather_s(s0 + 1, 1).start()
            gather_s(s0, 0).wait()

            @pl.when(t >= 1)
            def _():
                out_s(s0, 0).wait()
            transpose(0)
            out_s(s0, 0).start()

            s1 = s0 + 1

            @pl.when(s1 + 1 < SEQ_LEN)
            def _():
                gather_s(s1 + 1, 0).start()

            @pl.when(t >= 1)
            def _():
                out_s(s1, 1).wait()
            gather_s(s1, 1).wait()
            transpose(1)
            out_s(s1, 1).start()
            return carry

        lax.fori_loop(0, SEQ_LEN // 2, body, 0)
        out_s(SEQ_LEN - 2, 0).wait()
        out_s(SEQ_LEN - 1, 1).wait()

    return gather_kernel


_gather = _make_kernel()


@jax.jit
def kernel(indices, table):
    idx_t = jnp.swapaxes(indices, 0, 1).astype(jnp.int32)
    out5 = _gather(idx_t, table)
    return out5.transpose(2, 4, 0, 1, 3).reshape(BATCH, SEQ_LEN, EMB_DIM)


# speedup vs baseline: 1.2476x; 1.0792x over previous
import functools

import jax
import jax.numpy as jnp
from jax import lax
from jax.experimental import pallas as pl
from jax.experimental.pallas import tpu as pltpu
from jax.experimental.pallas import tpu_sc as plsc

VOCAB = 1000000
EMB_DIM = 32
BATCH = 4096
SEQ_LEN = 50

NC, NS = 2, 16
NW = NC * NS
BTILE = BATCH // NW
NGRP = EMB_DIM // 8


def _make_kernel():
    mesh = plsc.VectorSubcoreMesh(core_axis_name="c", subcore_axis_name="s")

    @functools.partial(
        pl.kernel,
        mesh=mesh,
        out_type=jax.ShapeDtypeStruct((SEQ_LEN, NGRP, NW, 8, BTILE),
                                      jnp.float32),
        scratch_types=[
            pltpu.VMEM((SEQ_LEN, BTILE), jnp.int32),
            pltpu.VMEM((2, BTILE, EMB_DIM), jnp.float32),
            pltpu.VMEM((2, NGRP, 8, BTILE), jnp.float32),
            [pltpu.SemaphoreType.DMA] * 2,
            [pltpu.SemaphoreType.DMA] * 2,
        ],
        compiler_params=pltpu.CompilerParams(use_tc_tiling_on_sc=False,
                                             needs_layout_passes=False),
    )
    def gather_kernel(idx_hbm, table_hbm, out_hbm, idx_v, rows_v, out_v,
                      gsem, osem):
        w = lax.axis_index("s") * NC + lax.axis_index("c")
        b0 = w * BTILE
        pltpu.sync_copy(idx_hbm.at[:, pl.ds(b0, BTILE)], idx_v)

        lane = lax.iota(jnp.int32, 16)
        row_ids = [lane + (16 * h) for h in range(BTILE // 16)]

        def gather_s(s, buf):
            return pltpu.make_async_copy(
                table_hbm.at[idx_v.at[s]], rows_v.at[buf], gsem[buf])

        def out_s(s, buf):
            return pltpu.make_async_copy(
                out_v.at[buf], out_hbm.at[s, :, w], osem[buf])

        def transpose(buf):
            for g in range(NGRP):
                for ci2 in range(0, 8, 2):
                    vals = []
                    for ci in (ci2, ci2 + 1):
                        col = jnp.full((16,), g * 8 + ci, jnp.int32)
                        for h in range(BTILE // 16):
                            vals.append(plsc.load_gather(
                                rows_v.at[buf], [row_ids[h], col]))
                    for k, ci in enumerate((ci2, ci2 + 1)):
                        for h in range(BTILE // 16):
                            out_v[buf, g, ci, pl.ds(16 * h, 16)] = (
                                vals[k * 8 + h])

        gather_s(0, 0).start()

        def body(t, carry):
            s0 = 2 * t
            gather_s(s0 + 1, 1).start()
            gather_s(s0, 0).wait()

            @pl.when(t >= 1)
            def _():
                out_s(s0, 0).wait()
            transpose(0)
            out_s(s0, 0).start()

            s1 = s0 + 1

            @pl.when(s1 + 1 < SEQ_LEN)
            def _():
                gather_s(s1 + 1, 0).start()

            @pl.when(t >= 1)
            def _():
                out_s(s1, 1).wait()
            gather_s(s1, 1).wait()
            transpose(1)
            out_s(s1, 1).start()
            return carry

        lax.fori_loop(0, SEQ_LEN // 2, body, 0)
        out_s(SEQ_LEN - 2, 0).wait()
        out_s(SEQ_LEN - 1, 1).wait()

    return gather_kernel


_gather = _make_kernel()


@jax.jit
def kernel(indices, table):
    idx_t = jnp.swapaxes(indices, 0, 1).astype(jnp.int32)
    out5 = _gather(idx_t, table)
    return out5.transpose(2, 4, 0, 1, 3).reshape(BATCH, SEQ_LEN, EMB_DIM)


# speedup vs baseline: 1.2501x; 1.0020x over previous
import functools

import jax
import jax.numpy as jnp
from jax import lax
from jax.experimental import pallas as pl
from jax.experimental.pallas import tpu as pltpu
from jax.experimental.pallas import tpu_sc as plsc

VOCAB = 1000000
EMB_DIM = 32
BATCH = 4096
SEQ_LEN = 50

NC, NS = 2, 16
NW = NC * NS
BTILE = BATCH // NW
NGRP = EMB_DIM // 8
NH = BTILE // 16


def _make_kernel():
    mesh = plsc.VectorSubcoreMesh(core_axis_name="c", subcore_axis_name="s")

    @functools.partial(
        pl.kernel,
        mesh=mesh,
        out_type=jax.ShapeDtypeStruct((SEQ_LEN, NGRP, NW, 8, BTILE),
                                      jnp.float32),
        scratch_types=[
            pltpu.VMEM((SEQ_LEN, BTILE), jnp.int32),
            pltpu.VMEM((2, BTILE), jnp.int32),
            pltpu.VMEM((2, BTILE), jnp.int32),
            pltpu.VMEM((2, BTILE, 128), jnp.float32),
            pltpu.VMEM((2, NGRP, 8, BTILE), jnp.float32),
            [pltpu.SemaphoreType.DMA] * 2,
            [pltpu.SemaphoreType.DMA] * 2,
        ],
        compiler_params=pltpu.CompilerParams(use_tc_tiling_on_sc=True,
                                             needs_layout_passes=False),
    )
    def gather_kernel(idx_hbm, tview_hbm, out_hbm, idx_v, j_v, q_v,
                      rows_v, out_v, gsem, osem):
        w = lax.axis_index("s") * NC + lax.axis_index("c")
        b0 = w * BTILE
        pltpu.sync_copy(idx_hbm.at[:, pl.ds(b0, BTILE)], idx_v)

        lane = lax.iota(jnp.int32, 16)

        def prep(s, buf):
            for h in range(NH):
                v = idx_v[s, pl.ds(16 * h, 16)]
                j_v[buf, pl.ds(16 * h, 16)] = v >> 2
                q_v[buf, pl.ds(16 * h, 16)] = (v & 3) << 5

        def gather_s(buf):
            return pltpu.make_async_copy(
                tview_hbm.at[j_v.at[buf]], rows_v.at[buf], gsem[buf])

        def out_s(s, buf):
            return pltpu.make_async_copy(
                out_v.at[buf], out_hbm.at[s, :, w], osem[buf])

        def transpose(buf):
            rows = [lane + 16 * h for h in range(NH)]
            qs = [q_v[buf, pl.ds(16 * h, 16)] for h in range(NH)]
            for g in range(NGRP):
                for ci2 in range(0, 8, 2):
                    vals = []
                    for ci in (ci2, ci2 + 1):
                        for h in range(NH):
                            vals.append(plsc.load_gather(
                                rows_v.at[buf],
                                [rows[h], qs[h] + (g * 8 + ci)]))
                    for k, ci in enumerate((ci2, ci2 + 1)):
                        for h in range(NH):
                            out_v[buf, g, ci, pl.ds(16 * h, 16)] = (
                                vals[k * 8 + h])

        prep(0, 0)
        gather_s(0).start()

        def body(t, carry):
            s0 = 2 * t
            prep(s0 + 1, 1)
            gather_s(1).start()
            gather_s(0).wait()

            @pl.when(t >= 1)
            def _():
                out_s(s0, 0).wait()
            transpose(0)
            out_s(s0, 0).start()

            s1 = s0 + 1

            @pl.when(s1 + 1 < SEQ_LEN)
            def _():
                prep(s1 + 1, 0)
                gather_s(0).start()

            @pl.when(t >= 1)
            def _():
                out_s(s1, 1).wait()
            gather_s(1).wait()
            transpose(1)
            out_s(s1, 1).start()
            return carry

        lax.fori_loop(0, SEQ_LEN // 2, body, 0)
        out_s(SEQ_LEN - 2, 0).wait()
        out_s(SEQ_LEN - 1, 1).wait()

    return gather_kernel


_gather = _make_kernel()


@jax.jit
def kernel(indices, table):
    idx_t = jnp.swapaxes(indices, 0, 1).astype(jnp.int32)
    tview = table.reshape(VOCAB // 4, 4 * EMB_DIM)
    out5 = _gather(idx_t, tview)
    return out5.transpose(2, 4, 0, 1, 3).reshape(BATCH, SEQ_LEN, EMB_DIM)
